# baseline (device time: 76076 ns/iter reference)
import math

import jax
import jax.numpy as jnp
from jax import lax
from jax.experimental import pallas as pl
from jax.experimental.pallas import tpu as pltpu

N_DEV = 4
B, SQ, D = 2, 512, 1024
H, DH = 8, 128
ROWS = B * SQ
SCALE = 0.08838834764831843
N_CH = 4
CH = SQ // N_CH


def _body(x_ref, wq_ref, wk_ref, wv_ref, wo_ref, out_ref,
          q_s, k_s, v_s, ctx_s, bufa, bufb, rsa, rsb, aga, agb,
          send_a, recv_a, send_b, recv_b):
    my = lax.axis_index("i")
    left = lax.rem(my + N_DEV - 1, N_DEV)
    right = lax.rem(my + 1, N_DEV)

    barrier = pltpu.get_barrier_semaphore()
    for nbr in (left, right):
        pl.semaphore_signal(barrier, inc=1, device_id=(nbr,),
                            device_id_type=pl.DeviceIdType.MESH)
    pl.semaphore_wait(barrier, 2)

    row = lax.broadcasted_iota(jnp.int32, (SQ, D), 0)
    col = lax.broadcasted_iota(jnp.int32, (SQ, D), 1)
    s_pos = row.astype(jnp.float32)
    d_in_head = lax.rem(col, DH)
    pair = (d_in_head // 2).astype(jnp.float32)
    inv = jnp.exp(pair * (-2.0 * math.log(10000.0) / DH))
    ang = s_pos * inv
    cos_t = jnp.cos(ang)
    sin_t = jnp.sin(ang)
    is_even = lax.rem(col, 2) == 0

    def rope(t):
        t_next = pltpu.roll(t, D - 1, 1)
        t_prev = pltpu.roll(t, 1, 1)
        t_rot = jnp.where(is_even, -t_next, t_prev)
        return t * cos_t + t_rot * sin_t

    wq = wq_ref[...].astype(jnp.bfloat16)
    wk = wk_ref[...].astype(jnp.bfloat16)
    wv = wv_ref[...].astype(jnp.bfloat16)
    for b in range(B):
        r0 = b * SQ
        xb = x_ref[r0:r0 + SQ, :].astype(jnp.bfloat16)
        q = jnp.dot(xb, wq, preferred_element_type=jnp.float32)
        q_s[r0:r0 + SQ, :] = rope(q).astype(jnp.bfloat16)
        k = jnp.dot(xb, wk, preferred_element_type=jnp.float32)
        k_s[r0:r0 + SQ, :] = rope(k).astype(jnp.bfloat16)
        v = jnp.dot(xb, wv, preferred_element_type=jnp.float32)
        v_s[r0:r0 + SQ, :] = v.astype(jnp.bfloat16)

    wo = wo_ref[...].astype(jnp.bfloat16)
    mod = lambda v: lax.rem(v + 4 * N_DEV, N_DEV)

    rings = (
        (bufa, rsa, aga, send_a, recv_a, right, 1, 0),
        (bufb, rsb, agb, send_b, recv_b, left, -1, SQ),
    )

    def compute_chunk(ring, c):
        buf, _, _, _, _, _, _, off = rings[ring]
        row = off + c * CH
        for h8 in range(H):
            c0 = h8 * DH
            qb = q_s[pl.ds(row, CH), pl.ds(c0, DH)]
            kb = k_s[pl.ds(off, SQ), pl.ds(c0, DH)]
            s = lax.dot_general(qb, kb, (((1,), (1,)), ((), ())),
                                preferred_element_type=jnp.float32) * SCALE
            m = jnp.max(s, axis=1, keepdims=True)
            w = jnp.exp(s - m)
            w = w / jnp.sum(w, axis=1, keepdims=True)
            cb = jnp.dot(w.astype(jnp.bfloat16),
                         v_s[pl.ds(off, SQ), pl.ds(c0, DH)],
                         preferred_element_type=jnp.float32)
            ctx_s[pl.ds(row, CH), pl.ds(c0, DH)] = cb.astype(jnp.bfloat16)
        ph = jnp.dot(ctx_s[pl.ds(row, CH), :], wo,
                     preferred_element_type=jnp.float32)
        buf[c, :, :] = ph.astype(jnp.bfloat16)

    def rs_desc(ring, h):
        buf, rs_recv, _, ssem, rsem, dst_dev, sign, _ = rings[ring]
        return pltpu.make_async_remote_copy(
            src_ref=buf.at[mod(my - sign * h)],
            dst_ref=rs_recv.at[h],
            send_sem=ssem.at[h], recv_sem=rsem.at[h],
            device_id=(dst_dev,), device_id_type=pl.DeviceIdType.MESH)

    def ag_desc(ring, h):
        buf, _, ag_recv, ssem, rsem, dst_dev, sign, _ = rings[ring]
        src = buf.at[mod(my + sign)] if h == 0 else ag_recv.at[h - 1]
        return pltpu.make_async_remote_copy(
            src_ref=src, dst_ref=ag_recv.at[h],
            send_sem=ssem.at[h + 3], recv_sem=rsem.at[h + 3],
            device_id=(dst_dev,), device_id_type=pl.DeviceIdType.MESH)

    def rs_acc(ring, h):
        buf, rs_recv, _, _, _, _, sign, off = rings[ring]
        c_r = mod(my - sign * (1 + h))
        s = (rs_recv[h, :, :].astype(jnp.float32)
             + buf[c_r, :, :].astype(jnp.float32))
        if h == N_DEV - 2:
            out_ref[pl.ds(off + c_r * CH, CH), :] = s
        buf[c_r, :, :] = s.astype(jnp.bfloat16)

    def ag_store(ring, h):
        _, _, ag_recv, _, _, _, sign, off = rings[ring]
        c_r = mod(my - sign * h)
        out_ref[pl.ds(off + c_r * CH, CH), :] = (
            ag_recv[h, :, :].astype(jnp.float32))

    compute_chunk(0, mod(my))
    dA = rs_desc(0, 0); dA.start()
    compute_chunk(1, mod(my))
    dB = rs_desc(1, 0); dB.start()
    for h in range(N_DEV - 2):
        compute_chunk(0, mod(my - 1 - h))
        dA.wait(); rs_acc(0, h); dA = rs_desc(0, h + 1); dA.start()
        compute_chunk(1, mod(my + 1 + h))
        dB.wait(); rs_acc(1, h); dB = rs_desc(1, h + 1); dB.start()
    compute_chunk(0, mod(my + 1))
    dA.wait(); rs_acc(0, N_DEV - 2); aA = ag_desc(0, 0); aA.start()
    compute_chunk(1, mod(my - 1))
    dB.wait(); rs_acc(1, N_DEV - 2); aB = ag_desc(1, 0); aB.start()

    for h in range(N_DEV - 1):
        aA.wait(); ag_store(0, h)
        if h < N_DEV - 2:
            nA = ag_desc(0, h + 1); nA.start()
        aB.wait(); ag_store(1, h)
        if h < N_DEV - 2:
            nB = ag_desc(1, h + 1); nB.start()
            aA, aB = nA, nB


def kernel(x, Wq, Wk, Wv, Wo):
    x2 = x.reshape(ROWS, D)
    out = pl.pallas_call(
        _body,
        out_shape=jax.ShapeDtypeStruct((ROWS, D), jnp.float32),
        in_specs=[pl.BlockSpec(memory_space=pltpu.VMEM)] * 5,
        out_specs=pl.BlockSpec(memory_space=pltpu.VMEM),
        scratch_shapes=[
            pltpu.VMEM((ROWS, D), jnp.bfloat16),
            pltpu.VMEM((ROWS, D), jnp.bfloat16),
            pltpu.VMEM((ROWS, D), jnp.bfloat16),
            pltpu.VMEM((ROWS, D), jnp.bfloat16),
            pltpu.VMEM((N_CH, CH, D), jnp.bfloat16),
            pltpu.VMEM((N_CH, CH, D), jnp.bfloat16),
            pltpu.VMEM((N_DEV - 1, CH, D), jnp.bfloat16),
            pltpu.VMEM((N_DEV - 1, CH, D), jnp.bfloat16),
            pltpu.VMEM((N_DEV - 1, CH, D), jnp.bfloat16),
            pltpu.VMEM((N_DEV - 1, CH, D), jnp.bfloat16),
            pltpu.SemaphoreType.DMA((6,)),
            pltpu.SemaphoreType.DMA((6,)),
            pltpu.SemaphoreType.DMA((6,)),
            pltpu.SemaphoreType.DMA((6,)),
        ],
        compiler_params=pltpu.CompilerParams(collective_id=0),
    )(x2, Wq, Wk, Wv, Wo)
    return out.reshape(B, SQ, D)


# device time: 59130 ns/iter; 1.2866x vs baseline; 1.2866x over previous
import math

import jax
import jax.numpy as jnp
from jax import lax
from jax.experimental import pallas as pl
from jax.experimental.pallas import tpu as pltpu

N_DEV = 4
B, SQ, D = 2, 512, 1024
H, DH = 8, 128
ROWS = B * SQ
SCALE = 0.08838834764831843
N_CH = 4
CH = SQ // 2 // N_CH


def _body(x_ref, wq_ref, wk_ref, wv_ref, wo_ref, out_ref,
          q_s, k_s, v_s, buf0, buf1, buf2, buf3, rs0, rs1, rs2, rs3,
          sem_sr, sem_rr, sem_sl, sem_rl):
    my = lax.axis_index("i")
    left = lax.rem(my + N_DEV - 1, N_DEV)
    right = lax.rem(my + 1, N_DEV)

    barrier = pltpu.get_barrier_semaphore()
    for nbr in (left, right):
        pl.semaphore_signal(barrier, inc=1, device_id=(nbr,),
                            device_id_type=pl.DeviceIdType.MESH)
    pl.semaphore_wait(barrier, 2)

    row = lax.broadcasted_iota(jnp.int32, (SQ, D), 0)
    col = lax.broadcasted_iota(jnp.int32, (SQ, D), 1)
    s_pos = row.astype(jnp.float32)
    d_in_head = lax.rem(col, DH)
    pair = (d_in_head // 2).astype(jnp.float32)
    inv = jnp.exp(pair * (-2.0 * math.log(10000.0) / DH))
    ang = s_pos * inv
    cos_t = jnp.cos(ang).astype(jnp.bfloat16)
    sin_t = jnp.sin(ang).astype(jnp.bfloat16)
    is_even = lax.rem(col, 2) == 0

    def rope(t):
        t_next = pltpu.roll(t, D - 1, 1)
        t_prev = pltpu.roll(t, 1, 1)
        t_rot = jnp.where(is_even, -t_next, t_prev)
        return (t * cos_t.astype(jnp.float32)
                + t_rot * sin_t.astype(jnp.float32))

    def qkv(b):
        r0 = b * SQ
        xb = x_ref[r0:r0 + SQ, :].astype(jnp.bfloat16)
        q = jnp.dot(xb, wq_ref[...].astype(jnp.bfloat16),
                    preferred_element_type=jnp.float32)
        q_s[r0:r0 + SQ, :] = rope(q).astype(jnp.bfloat16)
        k = jnp.dot(xb, wk_ref[...].astype(jnp.bfloat16),
                    preferred_element_type=jnp.float32)
        k_s[r0:r0 + SQ, :] = rope(k).astype(jnp.bfloat16)
        v = jnp.dot(xb, wv_ref[...].astype(jnp.bfloat16),
                    preferred_element_type=jnp.float32)
        v_s[r0:r0 + SQ, :] = v.astype(jnp.bfloat16)

    wo = wo_ref[...].astype(jnp.bfloat16)
    mod = lambda v: lax.rem(v + 4 * N_DEV, N_DEV)

    subrings = (
        (buf0, rs0, sem_sr, sem_rr, right, 1, 0, 0),
        (buf1, rs1, sem_sl, sem_rl, left, -1, 256, 0),
        (buf2, rs2, sem_sr, sem_rr, right, 1, 512, 6),
        (buf3, rs3, sem_sl, sem_rl, left, -1, 768, 6),
    )
    pend = [None] * 4

    def rs_desc(r, h):
        buf, rsr, ssem, rsem, dst_dev, sign, _, so = subrings[r]
        return pltpu.make_async_remote_copy(
            src_ref=buf.at[mod(my - sign * h)], dst_ref=rsr.at[h % 2],
            send_sem=ssem.at[so + h], recv_sem=rsem.at[so + h],
            device_id=(dst_dev,), device_id_type=pl.DeviceIdType.MESH)

    def ag_desc(r, h):
        buf, rsr, ssem, rsem, dst_dev, sign, _, so = subrings[r]
        src = buf.at[mod(my + sign)] if h == 0 else rsr.at[h % 2]
        return pltpu.make_async_remote_copy(
            src_ref=src, dst_ref=rsr.at[(h + 1) % 2],
            send_sem=ssem.at[so + h + 3], recv_sem=rsem.at[so + h + 3],
            device_id=(dst_dev,), device_id_type=pl.DeviceIdType.MESH)

    def advance(r, step):
        buf, rsr, _, _, _, sign, base, _ = subrings[r]
        pend[r].wait()
        if step < 3:
            c_r = mod(my - sign * (1 + step))
            if step == 2:
                s = (rsr[step % 2, :, :].astype(jnp.float32)
                     + buf[c_r, :, :].astype(jnp.float32))
                out_ref[pl.ds(base + c_r * CH, CH), :] = s
                buf[c_r, :, :] = s.astype(jnp.bfloat16)
            else:
                buf[c_r, :, :] = rsr[step % 2, :, :] + buf[c_r, :, :]
            pend[r] = rs_desc(r, step + 1) if step < 2 else ag_desc(r, 0)
            pend[r].start()
        else:
            h = step - 3
            c_r = mod(my - sign * h)
            out_ref[pl.ds(base + c_r * CH, CH), :] = (
                rsr[(h + 1) % 2, :, :].astype(jnp.float32))
            if h < 2:
                pend[r] = ag_desc(r, h + 1)
                pend[r].start()

    def head(b, h, ph):
        r0 = b * SQ
        c0 = h * DH
        qb = q_s[r0:r0 + SQ, c0:c0 + DH]
        kb = k_s[r0:r0 + SQ, c0:c0 + DH]
        s = lax.dot_general(qb, kb, (((1,), (1,)), ((), ())),
                            preferred_element_type=jnp.float32)
        e = jnp.exp(s * SCALE)
        denom = jnp.sum(e, axis=1, keepdims=True)
        cb = jnp.dot(e.astype(jnp.bfloat16), v_s[r0:r0 + SQ, c0:c0 + DH],
                     preferred_element_type=jnp.float32)
        cb = cb / denom
        return ph + jnp.dot(cb.astype(jnp.bfloat16), wo[c0:c0 + DH, :],
                            preferred_element_type=jnp.float32)

    def fill(b):
        phb = ph.astype(jnp.bfloat16)
        br, bl = (buf0, buf1) if b == 0 else (buf2, buf3)
        for c in range(N_CH):
            br[c, :, :] = phb[c * CH:(c + 1) * CH, :]
            bl[c, :, :] = phb[256 + c * CH:256 + (c + 1) * CH, :]

    for b in range(B):
        qkv(b)
        ph = jnp.zeros((SQ, D), jnp.float32)
        for h8 in range(H):
            ph = head(b, h8, ph)
        fill(b)
        for r in (2 * b, 2 * b + 1):
            pend[r] = rs_desc(r, 0)
            pend[r].start()

    for step in range(6):
        for r in range(4):
            advance(r, step)


def kernel(x, Wq, Wk, Wv, Wo):
    x2 = x.reshape(ROWS, D)
    out = pl.pallas_call(
        _body,
        out_shape=jax.ShapeDtypeStruct((ROWS, D), jnp.float32),
        in_specs=[pl.BlockSpec(memory_space=pltpu.VMEM)] * 5,
        out_specs=pl.BlockSpec(memory_space=pltpu.VMEM),
        scratch_shapes=[
            pltpu.VMEM((ROWS, D), jnp.bfloat16),
            pltpu.VMEM((ROWS, D), jnp.bfloat16),
            pltpu.VMEM((ROWS, D), jnp.bfloat16),
            pltpu.VMEM((N_CH, CH, D), jnp.bfloat16),
            pltpu.VMEM((N_CH, CH, D), jnp.bfloat16),
            pltpu.VMEM((N_CH, CH, D), jnp.bfloat16),
            pltpu.VMEM((N_CH, CH, D), jnp.bfloat16),
            pltpu.VMEM((2, CH, D), jnp.bfloat16),
            pltpu.VMEM((2, CH, D), jnp.bfloat16),
            pltpu.VMEM((2, CH, D), jnp.bfloat16),
            pltpu.VMEM((2, CH, D), jnp.bfloat16),
            pltpu.SemaphoreType.DMA((12,)),
            pltpu.SemaphoreType.DMA((12,)),
            pltpu.SemaphoreType.DMA((12,)),
            pltpu.SemaphoreType.DMA((12,)),
        ],
        compiler_params=pltpu.CompilerParams(collective_id=0),
    )(x2, Wq, Wk, Wv, Wo)
    return out.reshape(B, SQ, D)
